# baseline (speedup 1.0000x reference)
import functools

import jax
import jax.numpy as jnp
from jax import lax
from jax.experimental import pallas as pl
from jax.experimental.pallas import tpu as pltpu
from jax.experimental.pallas import tpu_sc as plsc

_VOCAB = 32100
_D = 32
_L = 50
_B = 4096
_H = 2048
_NC, _NS = 2, 16
_NW = _NC * _NS
_ROWS = 2 * _B
_RPW = _ROWS // _NW
_CH = 16
_NCHUNK = _RPW // _CH
_IDS = _CH * _L
_NGRP = 8
_GRP = _IDS // _NGRP
_TROWS = 2008


_sc_mesh = plsc.VectorSubcoreMesh(core_axis_name="c", subcore_axis_name="s")


@functools.partial(
    pl.kernel,
    out_type=jax.ShapeDtypeStruct((_ROWS, _D), jnp.float32),
    mesh=_sc_mesh,
    scratch_types=[
        pltpu.VMEM((2, _NGRP, _GRP), jnp.int32),
        pltpu.VMEM((2, _IDS, _D), jnp.float32),
        pltpu.VMEM((_CH, _D), jnp.float32),
        pltpu.VMEM_SHARED((_VOCAB, _D), jnp.float32),
        pltpu.SemaphoreType.DMA,
        pltpu.SemaphoreType.DMA,
    ],
    compiler_params=pltpu.CompilerParams(use_tc_tiling_on_sc=False),
)
def _sc_pool(emb_hbm, ids_hbm, out_hbm, idx_v, rows_v, acc_v, tab_spm, sem0, sem1):
    cid = lax.axis_index("c")
    sid = lax.axis_index("s")
    wid = sid * _NC + cid
    row0 = wid * _RPW
    sems = (sem0, sem1)

    t0 = jnp.minimum(sid * _TROWS, _VOCAB - _TROWS)
    pltpu.sync_copy(emb_hbm.at[pl.ds(t0, _TROWS)], tab_spm.at[pl.ds(t0, _TROWS)])
    plsc.subcore_barrier()

    def start(t, buf):
        pltpu.sync_copy(ids_hbm.at[wid, t], idx_v.at[buf])
        return [
            pltpu.async_copy(
                tab_spm.at[idx_v.at[buf, g]],
                rows_v.at[buf, pl.ds(g * _GRP, _GRP)],
                sems[buf],
            )
            for g in range(_NGRP)
        ]

    def accumulate(buf):
        def row_body(r, carry):
            base = r * _L
            a0 = jnp.zeros((16,), jnp.float32)
            a1 = jnp.zeros((16,), jnp.float32)
            for l in range(_L):
                a0 = a0 + rows_v[buf, base + l, 0:16]
                a1 = a1 + rows_v[buf, base + l, 16:32]
            acc_v[r, 0:16] = a0
            acc_v[r, 16:32] = a1
            return carry
        lax.fori_loop(0, _CH, row_body, 0, unroll=False)

    handles = start(0, 0)
    for t in range(_NCHUNK):
        buf = t & 1
        for h in handles:
            h.wait()
        if t + 1 < _NCHUNK:
            handles = start(t + 1, (t + 1) & 1)
        accumulate(buf)
        pltpu.sync_copy(acc_v, out_hbm.at[pl.ds(row0 + t * _CH, _CH)])


_BT = 256


def _tc_dense_body(x_ref, w_ref, b_ref, out_ref):
    o = jnp.dot(x_ref[0], w_ref[0], preferred_element_type=jnp.float32)
    out_ref[0] = jnp.maximum(o + b_ref[0], 0.0)


_tc_dense = pl.pallas_call(
    _tc_dense_body,
    grid=(3, _B // _BT),
    in_specs=[
        pl.BlockSpec((1, _BT, 64), lambda j, b: (j, b, 0)),
        pl.BlockSpec((1, 64, _H), lambda j, b: (j, 0, 0)),
        pl.BlockSpec((1, 1, _H), lambda j, b: (j, 0, 0)),
    ],
    out_specs=pl.BlockSpec((1, _BT, _H), lambda j, b: (j + 2, b, 0)),
    out_shape=jax.ShapeDtypeStruct((5, _B, _H), jnp.float32),
)


def _tc_anchor_body(big_ref, pooled_ref, aW_ref, ab_ref, out_ref):
    del big_ref
    x = pooled_ref[0] * jnp.float32(1.0 / _L)
    o = jnp.dot(x, aW_ref[...], preferred_element_type=jnp.float32)
    out_ref[0] = jnp.maximum(o + ab_ref[...], 0.0)


_tc_anchor = pl.pallas_call(
    _tc_anchor_body,
    grid=(2, _B // _BT),
    in_specs=[
        pl.BlockSpec(memory_space=pl.ANY),
        pl.BlockSpec((1, _BT, _D), lambda j, b: (j, b, 0)),
        pl.BlockSpec((_D, _H), lambda j, b: (0, 0)),
        pl.BlockSpec((1, _H), lambda j, b: (0, 0)),
    ],
    out_specs=pl.BlockSpec((1, _BT, _H), lambda j, b: (j, b, 0)),
    out_shape=jax.ShapeDtypeStruct((5, _B, _H), jnp.float32),
    input_output_aliases={0: 0},
)


def kernel(anchor_out_ids, anchor_in_ids, domain_out, domain_in, numerics,
           emb, aW, ab, dW, db, nW, nb):
    ids_all = jnp.concatenate(
        [anchor_out_ids.reshape(-1), anchor_in_ids.reshape(-1)]
    ).astype(jnp.int32)
    ids_pre = ids_all.reshape(_NW, _NCHUNK, _NGRP, _GRP)

    x3 = jnp.stack(
        [domain_out.astype(jnp.float32), domain_in.astype(jnp.float32),
         jnp.pad(numerics, ((0, 0), (0, 64 - 8)))], axis=0)
    w3 = jnp.stack([dW, dW, jnp.pad(nW, ((0, 64 - 8), (0, 0)))], axis=0)
    b3 = jnp.stack([db, db, nb], axis=0).reshape(3, 1, _H)

    pooled = _sc_pool(emb, ids_pre)
    pooled3 = pooled.reshape(2, _B, _D)

    out_dense = _tc_dense(x3, w3, b3)
    out3 = _tc_anchor(out_dense, pooled3, aW, ab.reshape(1, _H))
    return jnp.transpose(out3, (1, 0, 2))


# speedup vs baseline: 1.0603x; 1.0603x over previous
import functools

import jax
import jax.numpy as jnp
from jax import lax
from jax.experimental import pallas as pl
from jax.experimental.pallas import tpu as pltpu
from jax.experimental.pallas import tpu_sc as plsc

_VOCAB = 32100
_D = 32
_L = 50
_B = 4096
_H = 2048
_NC, _NS = 2, 16
_NW = _NC * _NS
_ROWS = 2 * _B
_RPW = _ROWS // _NW
_CH = 16
_NCHUNK = _RPW // _CH
_IDS = _CH * _L
_NGRP = 8
_GRP = _IDS // _NGRP
_TROWS = 2008


_sc_mesh = plsc.VectorSubcoreMesh(core_axis_name="c", subcore_axis_name="s")


@functools.partial(
    pl.kernel,
    out_type=jax.ShapeDtypeStruct((_ROWS, _D), jnp.float32),
    mesh=_sc_mesh,
    scratch_types=[
        pltpu.VMEM((2, _NGRP, _GRP), jnp.int32),
        pltpu.VMEM((2, _IDS, _D // 2), jnp.int32),
        pltpu.VMEM((_CH, _D), jnp.float32),
        pltpu.VMEM_SHARED((_VOCAB, _D // 2), jnp.int32),
        pltpu.SemaphoreType.DMA,
        pltpu.SemaphoreType.DMA,
    ],
    compiler_params=pltpu.CompilerParams(use_tc_tiling_on_sc=False),
)
def _sc_pool(emb_hbm, ids_hbm, out_hbm, idx_v, rows_v, acc_v, tab_spm, sem0, sem1):
    cid = lax.axis_index("c")
    sid = lax.axis_index("s")
    wid = sid * _NC + cid
    row0 = wid * _RPW
    sems = (sem0, sem1)

    t0 = jnp.minimum(sid * _TROWS, _VOCAB - _TROWS)
    pltpu.sync_copy(emb_hbm.at[pl.ds(t0, _TROWS)], tab_spm.at[pl.ds(t0, _TROWS)])
    plsc.subcore_barrier()

    def start(t, buf):
        pltpu.sync_copy(ids_hbm.at[wid, t], idx_v.at[buf])
        return [
            pltpu.async_copy(
                tab_spm.at[idx_v.at[buf, g]],
                rows_v.at[buf, pl.ds(g * _GRP, _GRP)],
                sems[buf],
            )
            for g in range(_NGRP)
        ]

    def accumulate(buf):
        def row_body(r, carry):
            base = r * _L
            a0 = jnp.zeros((16,), jnp.float32)
            a1 = jnp.zeros((16,), jnp.float32)
            hi = jnp.int32(-65536)
            for l in range(_L):
                x = rows_v[buf, base + l, :]
                a0 = a0 + lax.bitcast_convert_type(jnp.left_shift(x, 16), jnp.float32)
                a1 = a1 + lax.bitcast_convert_type(jnp.bitwise_and(x, hi), jnp.float32)
            acc_v[r, 0:16] = a0
            acc_v[r, 16:32] = a1
            return carry
        lax.fori_loop(0, _CH, row_body, 0, unroll=False)

    handles = start(0, 0)
    for t in range(_NCHUNK):
        buf = t & 1
        for h in handles:
            h.wait()
        if t + 1 < _NCHUNK:
            handles = start(t + 1, (t + 1) & 1)
        accumulate(buf)
        pltpu.sync_copy(acc_v, out_hbm.at[pl.ds(row0 + t * _CH, _CH)])


_BT = 512


def _tc_dense_body(x_ref, w_ref, b_ref, out_ref):
    o = jnp.dot(x_ref[0], w_ref[0], preferred_element_type=jnp.float32)
    out_ref[0] = jnp.maximum(o + b_ref[0], 0.0)


_tc_dense = pl.pallas_call(
    _tc_dense_body,
    grid=(3, _B // _BT),
    in_specs=[
        pl.BlockSpec((1, _BT, 64), lambda j, b: (j, b, 0)),
        pl.BlockSpec((1, 64, _H), lambda j, b: (j, 0, 0)),
        pl.BlockSpec((1, 1, _H), lambda j, b: (j, 0, 0)),
    ],
    out_specs=pl.BlockSpec((1, _BT, _H), lambda j, b: (j + 2, b, 0)),
    out_shape=jax.ShapeDtypeStruct((5, _B, _H), jnp.float32),
)


def _tc_anchor_body(big_ref, pooled_ref, aW_ref, ab_ref, out_ref):
    del big_ref
    x = pooled_ref[0] * jnp.float32(1.0 / _L)
    o = jnp.dot(x, aW_ref[...], preferred_element_type=jnp.float32)
    out_ref[0] = jnp.maximum(o + ab_ref[...], 0.0)


_tc_anchor = pl.pallas_call(
    _tc_anchor_body,
    grid=(2, _B // _BT),
    in_specs=[
        pl.BlockSpec(memory_space=pl.ANY),
        pl.BlockSpec((1, _BT, _D), lambda j, b: (j, b, 0)),
        pl.BlockSpec((_D, _H), lambda j, b: (0, 0)),
        pl.BlockSpec((1, _H), lambda j, b: (0, 0)),
    ],
    out_specs=pl.BlockSpec((1, _BT, _H), lambda j, b: (j, b, 0)),
    out_shape=jax.ShapeDtypeStruct((5, _B, _H), jnp.float32),
    input_output_aliases={0: 0},
)


def kernel(anchor_out_ids, anchor_in_ids, domain_out, domain_in, numerics,
           emb, aW, ab, dW, db, nW, nb):
    ids_all = jnp.concatenate(
        [anchor_out_ids.reshape(-1), anchor_in_ids.reshape(-1)]
    ).astype(jnp.int32)
    ids_pre = ids_all.reshape(_NW, _NCHUNK, _NGRP, _GRP)

    x3 = jnp.stack(
        [domain_out.astype(jnp.float32), domain_in.astype(jnp.float32),
         jnp.pad(numerics, ((0, 0), (0, 64 - 8)))], axis=0)
    w3 = jnp.stack([dW, dW, jnp.pad(nW, ((0, 64 - 8), (0, 0)))], axis=0)
    b3 = jnp.stack([db, db, nb], axis=0).reshape(3, 1, _H)

    emb_i = lax.bitcast_convert_type(
        emb.astype(jnp.bfloat16).reshape(_VOCAB, _D // 2, 2), jnp.int32)
    pooled = _sc_pool(emb_i, ids_pre)
    pooled3 = pooled.reshape(2, _B, _D)
    perm = jnp.asarray(list(range(0, _D, 2)) + list(range(1, _D, 2)))
    aW_perm = aW[perm, :]

    out_dense = _tc_dense(x3, w3, b3)
    out3 = _tc_anchor(out_dense, pooled3, aW_perm, ab.reshape(1, _H))
    return jnp.transpose(out3, (1, 0, 2))


# speedup vs baseline: 1.2090x; 1.1402x over previous
import functools

import jax
import jax.numpy as jnp
from jax import lax
from jax.experimental import pallas as pl
from jax.experimental.pallas import tpu as pltpu
from jax.experimental.pallas import tpu_sc as plsc

_VOCAB = 32100
_D = 32
_L = 50
_B = 4096
_H = 2048
_NC, _NS = 2, 16
_NW = _NC * _NS
_ROWS = 2 * _B
_RPW = _ROWS // _NW
_CH = 16
_NCHUNK = _RPW // _CH
_IDS = _CH * _L
_NGRP = 8
_GRP = _IDS // _NGRP
_TROWS = 2008


_sc_mesh = plsc.VectorSubcoreMesh(core_axis_name="c", subcore_axis_name="s")


@functools.partial(
    pl.kernel,
    out_type=jax.ShapeDtypeStruct((_ROWS, _D), jnp.float32),
    mesh=_sc_mesh,
    scratch_types=[
        pltpu.VMEM((2, _NGRP, _GRP), jnp.int32),
        pltpu.VMEM((2, _IDS, _D), jnp.float32),
        pltpu.VMEM((_CH, _D), jnp.float32),
        pltpu.VMEM_SHARED((_VOCAB, _D), jnp.float32),
        pltpu.SemaphoreType.DMA,
        pltpu.SemaphoreType.DMA,
    ],
    compiler_params=pltpu.CompilerParams(use_tc_tiling_on_sc=False),
)
def _sc_pool(emb_hbm, ids_hbm, out_hbm, idx_v, rows_v, acc_v, tab_spm, sem0, sem1):
    cid = lax.axis_index("c")
    sid = lax.axis_index("s")
    wid = sid * _NC + cid
    row0 = wid * _RPW
    sems = (sem0, sem1)

    t0 = jnp.minimum(sid * _TROWS, _VOCAB - _TROWS)
    pltpu.sync_copy(emb_hbm.at[pl.ds(t0, _TROWS)], tab_spm.at[pl.ds(t0, _TROWS)])
    plsc.subcore_barrier()

    def start(t, buf):
        pltpu.sync_copy(ids_hbm.at[wid, t], idx_v.at[buf])
        return [
            pltpu.async_copy(
                tab_spm.at[idx_v.at[buf, g]],
                rows_v.at[buf, pl.ds(g * _GRP, _GRP)],
                sems[buf],
            )
            for g in range(_NGRP)
        ]

    def accumulate(buf):
        def row_body(r, carry):
            base = r * _L
            a0 = jnp.zeros((16,), jnp.float32)
            a1 = jnp.zeros((16,), jnp.float32)
            for l in range(_L):
                a0 = a0 + rows_v[buf, base + l, 0:16]
                a1 = a1 + rows_v[buf, base + l, 16:32]
            acc_v[r, 0:16] = a0
            acc_v[r, 16:32] = a1
            return carry
        lax.fori_loop(0, _CH, row_body, 0, unroll=False)

    handles = start(0, 0)
    for t in range(_NCHUNK):
        buf = t & 1
        for h in handles:
            h.wait()
        if t + 1 < _NCHUNK:
            handles = start(t + 1, (t + 1) & 1)
        accumulate(buf)
        pltpu.sync_copy(acc_v, out_hbm.at[pl.ds(row0 + t * _CH, _CH)])


_BT = 1024


def _tc_dense_body(x_ref, w_ref, b_ref, out_ref):
    o = jnp.dot(x_ref[0], w_ref[0], preferred_element_type=jnp.float32)
    out_ref[0] = jnp.maximum(o + b_ref[0], 0.0)


_tc_dense = pl.pallas_call(
    _tc_dense_body,
    grid=(3, _B // _BT),
    in_specs=[
        pl.BlockSpec((1, _BT, 64), lambda j, b: (j, b, 0)),
        pl.BlockSpec((1, 64, _H), lambda j, b: (j, 0, 0)),
        pl.BlockSpec((1, 1, _H), lambda j, b: (j, 0, 0)),
    ],
    out_specs=pl.BlockSpec((1, _BT, _H), lambda j, b: (j + 2, b, 0)),
    out_shape=jax.ShapeDtypeStruct((5, _B, _H), jnp.float32),
)


def _tc_anchor_body(big_ref, pooled_ref, aW_ref, ab_ref, out_ref):
    del big_ref
    x = pooled_ref[0] * jnp.float32(1.0 / _L)
    o = jnp.dot(x, aW_ref[...], preferred_element_type=jnp.float32)
    out_ref[0] = jnp.maximum(o + ab_ref[...], 0.0)


_tc_anchor = pl.pallas_call(
    _tc_anchor_body,
    grid=(2, _B // _BT),
    in_specs=[
        pl.BlockSpec(memory_space=pl.ANY),
        pl.BlockSpec((1, _BT, _D), lambda j, b: (j, b, 0)),
        pl.BlockSpec((_D, _H), lambda j, b: (0, 0)),
        pl.BlockSpec((1, _H), lambda j, b: (0, 0)),
    ],
    out_specs=pl.BlockSpec((1, _BT, _H), lambda j, b: (j, b, 0)),
    out_shape=jax.ShapeDtypeStruct((5, _B, _H), jnp.float32),
    input_output_aliases={0: 0},
)


def kernel(anchor_out_ids, anchor_in_ids, domain_out, domain_in, numerics,
           emb, aW, ab, dW, db, nW, nb):
    ids_all = jnp.concatenate(
        [anchor_out_ids.reshape(-1), anchor_in_ids.reshape(-1)]
    ).astype(jnp.int32)
    ids_pre = ids_all.reshape(_NW, _NCHUNK, _NGRP, _GRP)

    x3 = jnp.stack(
        [domain_out.astype(jnp.float32), domain_in.astype(jnp.float32),
         jnp.pad(numerics, ((0, 0), (0, 64 - 8)))], axis=0)
    w3 = jnp.stack([dW, dW, jnp.pad(nW, ((0, 64 - 8), (0, 0)))], axis=0)
    b3 = jnp.stack([db, db, nb], axis=0).reshape(3, 1, _H)

    pooled = _sc_pool(emb, ids_pre)
    pooled3 = pooled.reshape(2, _B, _D)

    out_dense = _tc_dense(x3, w3, b3)
    out3 = _tc_anchor(out_dense, pooled3, aW, ab.reshape(1, _H))
    return jnp.transpose(out3, (1, 0, 2))


# speedup vs baseline: 1.2236x; 1.0121x over previous
import functools

import jax
import jax.numpy as jnp
from jax import lax
from jax.experimental import pallas as pl
from jax.experimental.pallas import tpu as pltpu
from jax.experimental.pallas import tpu_sc as plsc

_VOCAB = 32100
_D = 32
_L = 50
_B = 4096
_H = 2048
_NC, _NS = 2, 16
_NW = _NC * _NS
_ROWS = 2 * _B
_RPW = _ROWS // _NW
_CH = 16
_NCHUNK = _RPW // _CH
_IDS = _CH * _L
_NGRP = 8
_GRP = _IDS // _NGRP
_TROWS = 2008


_sc_mesh = plsc.VectorSubcoreMesh(core_axis_name="c", subcore_axis_name="s")


@functools.partial(
    pl.kernel,
    out_type=jax.ShapeDtypeStruct((_ROWS, _D), jnp.float32),
    mesh=_sc_mesh,
    scratch_types=[
        pltpu.VMEM((2, _NGRP, _GRP), jnp.int32),
        pltpu.VMEM((2, _IDS, _D), jnp.float32),
        pltpu.VMEM((2, _CH, _D), jnp.float32),
        pltpu.VMEM_SHARED((_VOCAB, _D), jnp.float32),
        pltpu.SemaphoreType.DMA,
        pltpu.SemaphoreType.DMA,
        pltpu.SemaphoreType.DMA,
        pltpu.SemaphoreType.DMA,
    ],
    compiler_params=pltpu.CompilerParams(use_tc_tiling_on_sc=False),
)
def _sc_pool(emb_hbm, ids_hbm, out_hbm, idx_v, rows_v, acc_v, tab_spm,
             sem0, sem1, sem_ids, sem_out):
    cid = lax.axis_index("c")
    sid = lax.axis_index("s")
    wid = sid * _NC + cid
    row0 = wid * _RPW
    sems = (sem0, sem1)

    def ids_copy(t):
        return pltpu.async_copy(ids_hbm.at[wid, t], idx_v.at[t & 1], sem_ids)

    def gathers(t):
        buf = t & 1
        return [
            pltpu.async_copy(
                tab_spm.at[idx_v.at[buf, g]],
                rows_v.at[buf, pl.ds(g * _GRP, _GRP)],
                sems[buf],
            )
            for g in range(_NGRP)
        ]

    def accumulate(buf):
        def row_body(r, carry):
            base = r * _L
            a0 = jnp.zeros((16,), jnp.float32)
            a1 = jnp.zeros((16,), jnp.float32)
            for l in range(_L):
                a0 = a0 + rows_v[buf, base + l, 0:16]
                a1 = a1 + rows_v[buf, base + l, 16:32]
            acc_v[buf, r, 0:16] = a0
            acc_v[buf, r, 16:32] = a1
            return carry
        lax.fori_loop(0, _CH, row_body, 0, unroll=False)

    h_ids = ids_copy(0)
    t0 = jnp.minimum(sid * _TROWS, _VOCAB - _TROWS)
    pltpu.sync_copy(emb_hbm.at[pl.ds(t0, _TROWS)], tab_spm.at[pl.ds(t0, _TROWS)])
    plsc.subcore_barrier()

    h_ids.wait()
    handles = gathers(0)
    h_ids = ids_copy(1)
    h_out = []
    for t in range(_NCHUNK):
        buf = t & 1
        for h in handles:
            h.wait()
        if t + 1 < _NCHUNK:
            h_ids.wait()
            handles = gathers(t + 1)
            if t + 2 < _NCHUNK:
                h_ids = ids_copy(t + 2)
        if t >= 2:
            h_out[t - 2].wait()
        accumulate(buf)
        h_out.append(pltpu.async_copy(
            acc_v.at[buf], out_hbm.at[pl.ds(row0 + t * _CH, _CH)], sem_out))
    h_out[_NCHUNK - 2].wait()
    h_out[_NCHUNK - 1].wait()


_BT = 1024


def _tc_dense_body(x_ref, w_ref, b_ref, out_ref):
    o = jnp.dot(x_ref[0], w_ref[0], preferred_element_type=jnp.float32)
    out_ref[0] = jnp.maximum(o + b_ref[0], 0.0)


_tc_dense = pl.pallas_call(
    _tc_dense_body,
    grid=(3, _B // _BT),
    in_specs=[
        pl.BlockSpec((1, _BT, 64), lambda j, b: (j, b, 0)),
        pl.BlockSpec((1, 64, _H), lambda j, b: (j, 0, 0)),
        pl.BlockSpec((1, 1, _H), lambda j, b: (j, 0, 0)),
    ],
    out_specs=pl.BlockSpec((1, _BT, _H), lambda j, b: (j + 2, b, 0)),
    out_shape=jax.ShapeDtypeStruct((5, _B, _H), jnp.float32),
)


def _tc_anchor_body(big_ref, pooled_ref, aW_ref, ab_ref, out_ref):
    del big_ref
    x = pooled_ref[0] * jnp.float32(1.0 / _L)
    o = jnp.dot(x, aW_ref[...], preferred_element_type=jnp.float32)
    out_ref[0] = jnp.maximum(o + ab_ref[...], 0.0)


_tc_anchor = pl.pallas_call(
    _tc_anchor_body,
    grid=(2, _B // _BT),
    in_specs=[
        pl.BlockSpec(memory_space=pl.ANY),
        pl.BlockSpec((1, _BT, _D), lambda j, b: (j, b, 0)),
        pl.BlockSpec((_D, _H), lambda j, b: (0, 0)),
        pl.BlockSpec((1, _H), lambda j, b: (0, 0)),
    ],
    out_specs=pl.BlockSpec((1, _BT, _H), lambda j, b: (j, b, 0)),
    out_shape=jax.ShapeDtypeStruct((5, _B, _H), jnp.float32),
    input_output_aliases={0: 0},
)


def kernel(anchor_out_ids, anchor_in_ids, domain_out, domain_in, numerics,
           emb, aW, ab, dW, db, nW, nb):
    ids_all = jnp.concatenate(
        [anchor_out_ids.reshape(-1), anchor_in_ids.reshape(-1)]
    ).astype(jnp.int32)
    ids_pre = ids_all.reshape(_NW, _NCHUNK, _NGRP, _GRP)

    x3 = jnp.stack(
        [domain_out.astype(jnp.float32), domain_in.astype(jnp.float32),
         jnp.pad(numerics, ((0, 0), (0, 64 - 8)))], axis=0)
    w3 = jnp.stack([dW, dW, jnp.pad(nW, ((0, 64 - 8), (0, 0)))], axis=0)
    b3 = jnp.stack([db, db, nb], axis=0).reshape(3, 1, _H)

    pooled = _sc_pool(emb, ids_pre)
    pooled3 = pooled.reshape(2, _B, _D)

    out_dense = _tc_dense(x3, w3, b3)
    out3 = _tc_anchor(out_dense, pooled3, aW, ab.reshape(1, _H))
    return jnp.transpose(out3, (1, 0, 2))
